# Initial kernel scaffold; baseline (speedup 1.0000x reference)
import jax
import jax.numpy as jnp
from jax.experimental import pallas as pl


def kernel(h, pos, batch, W0, b0, g0, be0, W1, b1, g1, be1, W2, b2, g2, be2):
    raise NotImplementedError("write your pallas kernel here")


# speedup vs baseline: 1.5174x; 1.5174x over previous
import jax
import jax.numpy as jnp
from jax.experimental import pallas as pl
from jax.experimental.pallas import tpu as pltpu

_B, _P, _M, _NS = 16, 2048, 512, 64
_R2 = 0.2 * 0.2
_EPS = 1e-5
_QB = 256
_G = (_B * _M) // _QB
_RB = _QB * _NS
_E = _B * _M * _NS


def _graph(pos):
    pos3 = pos.reshape(_B, _P, 3)
    ar = jnp.arange(_B)

    def body(i, state):
        idx, dists, far = state
        idx = idx.at[:, i].set(far)
        c = pos3[ar, far]
        d = ((pos3 - c[:, None, :]) ** 2).sum(-1)
        dists = jnp.minimum(dists, d)
        far = dists.argmax(-1)
        return idx, dists, far

    idx0 = jnp.zeros((_B, _M), jnp.int32)
    dists0 = jnp.full((_B, _P), jnp.inf, jnp.float32)
    far0 = jnp.zeros((_B,), jnp.int32)
    idx, _, _ = jax.lax.fori_loop(0, _M, body, (idx0, dists0, far0))
    sampled = pos3[ar[:, None], idx]
    d2 = ((sampled[:, :, None, :] - pos3[:, None, :, :]) ** 2).sum(-1)
    within = d2 <= _R2
    order = jnp.argsort(~within, axis=-1, stable=True)
    nb = order[:, :, :_NS]
    cnt = jnp.minimum(within.sum(-1), _NS)
    return idx, nb, cnt


def _tw_kernel(hp_ref, w_ref, out_ref):
    out_ref[...] = jnp.dot(hp_ref[...], w_ref[...],
                           preferred_element_type=jnp.float32)


def _matmul_tw(hp, w):
    rows = hp.shape[0]
    blk = 2048
    return pl.pallas_call(
        _tw_kernel,
        grid=(rows // blk,),
        in_specs=[
            pl.BlockSpec((blk, hp.shape[1]), lambda i: (i, 0)),
            pl.BlockSpec((hp.shape[1], w.shape[1]), lambda i: (0, 0)),
        ],
        out_specs=pl.BlockSpec((blk, w.shape[1]), lambda i: (i, 0)),
        out_shape=jax.ShapeDtypeStruct((rows, w.shape[1]), jnp.float32),
    )(hp, w)


def _stats0_kernel(x0_ref, qw_ref, vf_ref, out_ref):
    x0 = x0_ref[...].reshape(_QB, _NS, 64) - qw_ref[...][:, None, :]
    w = vf_ref[...][:, :, None]
    xw = (x0 * w).reshape(_RB, 64)
    s1 = jnp.sum(xw, axis=0)
    s2 = jnp.sum(xw * x0.reshape(_RB, 64), axis=0)

    @pl.when(pl.program_id(0) == 0)
    def _():
        out_ref[...] = jnp.zeros_like(out_ref)

    out_ref[0:1, :] += s1[None]
    out_ref[1:2, :] += s2[None]


def _stats1_kernel(x0_ref, qw_ref, vf_ref, a0_ref, c0_ref, w1_ref, b1_ref,
                   out_ref):
    x0 = x0_ref[...].reshape(_QB, _NS, 64) - qw_ref[...][:, None, :]
    h0 = jnp.maximum(x0 * a0_ref[...][None] + c0_ref[...][None], 0.0)
    x1 = jnp.dot(h0.reshape(_RB, 64), w1_ref[...],
                 preferred_element_type=jnp.float32) + b1_ref[...]
    w = vf_ref[...][:, :, None]
    xw = (x1.reshape(_QB, _NS, 64) * w).reshape(_RB, 64)
    s1 = jnp.sum(xw, axis=0)
    s2 = jnp.sum(xw * x1, axis=0)

    @pl.when(pl.program_id(0) == 0)
    def _():
        out_ref[...] = jnp.zeros_like(out_ref)

    out_ref[0:1, :] += s1[None]
    out_ref[1:2, :] += s2[None]


def _stats2_kernel(x0_ref, qw_ref, vf_ref, a0_ref, c0_ref, w1_ref, b1_ref,
                   a1_ref, c1_ref, w2_ref, b2_ref, out_ref):
    x0 = x0_ref[...].reshape(_QB, _NS, 64) - qw_ref[...][:, None, :]
    h0 = jnp.maximum(x0 * a0_ref[...][None] + c0_ref[...][None], 0.0)
    x1 = jnp.dot(h0.reshape(_RB, 64), w1_ref[...],
                 preferred_element_type=jnp.float32) + b1_ref[...]
    h1 = jnp.maximum(x1 * a1_ref[...] + c1_ref[...], 0.0)
    x2 = jnp.dot(h1, w2_ref[...],
                 preferred_element_type=jnp.float32) + b2_ref[...]
    w = vf_ref[...][:, :, None]
    xw = (x2.reshape(_QB, _NS, 128) * w).reshape(_RB, 128)
    s1 = jnp.sum(xw, axis=0)
    s2 = jnp.sum(xw * x2, axis=0)

    @pl.when(pl.program_id(0) == 0)
    def _():
        out_ref[...] = jnp.zeros_like(out_ref)

    out_ref[0:1, :] += s1[None]
    out_ref[1:2, :] += s2[None]


def _final_kernel(x0_ref, qw_ref, vf_ref, a0_ref, c0_ref, w1_ref, b1_ref,
                  a1_ref, c1_ref, w2_ref, b2_ref, a2_ref, c2_ref, out_ref):
    x0 = x0_ref[...].reshape(_QB, _NS, 64) - qw_ref[...][:, None, :]
    h0 = jnp.maximum(x0 * a0_ref[...][None] + c0_ref[...][None], 0.0)
    x1 = jnp.dot(h0.reshape(_RB, 64), w1_ref[...],
                 preferred_element_type=jnp.float32) + b1_ref[...]
    h1 = jnp.maximum(x1 * a1_ref[...] + c1_ref[...], 0.0)
    x2 = jnp.dot(h1, w2_ref[...],
                 preferred_element_type=jnp.float32) + b2_ref[...]
    msg = jnp.maximum(x2 * a2_ref[...] + c2_ref[...], 0.0)
    vb = vf_ref[...][:, :, None] > 0.5
    msg = jnp.where(vb, msg.reshape(_QB, _NS, 128), -jnp.inf)
    r = jnp.max(msg, axis=1)
    out_ref[...] = jnp.where(jnp.isfinite(r), r, 0.0)


def _vec(x):
    return x.reshape(1, -1)


def _edge_pass(kern, n_out, extra, out_rows=None):
    def call(x0g, qw, vf):
        in_specs = [
            pl.BlockSpec((_RB, 64), lambda i: (i, 0)),
            pl.BlockSpec((_QB, 64), lambda i: (i, 0)),
            pl.BlockSpec((_QB, _NS), lambda i: (i, 0)),
        ]
        ops = [x0g, qw, vf]
        for e in extra:
            in_specs.append(
                pl.BlockSpec(e.shape, lambda i, r=len(e.shape): (0,) * r))
            ops.append(e)
        if out_rows is None:
            out_spec = pl.BlockSpec((8, n_out), lambda i: (0, 0))
            out_shape = jax.ShapeDtypeStruct((8, n_out), jnp.float32)
        else:
            out_spec = pl.BlockSpec((_QB, n_out), lambda i: (i, 0))
            out_shape = jax.ShapeDtypeStruct((out_rows, n_out), jnp.float32)
        return pl.pallas_call(
            kern,
            grid=(_G,),
            in_specs=in_specs,
            out_specs=out_spec,
            out_shape=out_shape,
        )(*ops)
    return call


def kernel(h, pos, batch, W0, b0, g0, be0, W1, b1, g1, be1, W2, b2, g2, be2):
    idx, nb, cnt = _graph(pos)
    ar = jnp.arange(_B)
    idx_g = (ar[:, None] * _P + idx).reshape(-1)
    src = (ar[:, None, None] * _P + nb).reshape(-1)

    ones = jnp.ones((_B * _P, 1), jnp.float32)
    zeros = jnp.zeros((_B * _P, 4), jnp.float32)
    hp = jnp.concatenate([h, pos, ones, zeros], axis=1)
    w0f = jnp.concatenate(
        [W0, b0[None, :], jnp.zeros((4, 64), jnp.float32)], axis=0)
    tw = _matmul_tw(hp, w0f)

    pos_dst = pos[idx_g]
    pd = jnp.concatenate([pos_dst, jnp.zeros((_B * _M, 5), jnp.float32)], 1)
    w0p = jnp.concatenate([W0[64:67], jnp.zeros((5, 64), jnp.float32)], 0)
    qw = _matmul_tw(pd, w0p)

    x0g = jnp.take(tw, src, axis=0)

    vf = (jnp.arange(_NS)[None, :] < cnt.reshape(-1)[:, None]
          ).astype(jnp.float32)
    n = cnt.sum().astype(jnp.float32)

    s0 = _edge_pass(_stats0_kernel, 64, [])(x0g, qw, vf)
    m0 = s0[0] / n
    v0 = s0[1] / n - m0 * m0
    a0 = _vec(g0 / jnp.sqrt(v0 + _EPS))
    c0 = _vec(be0 - m0 * (g0 / jnp.sqrt(v0 + _EPS)))

    s1 = _edge_pass(_stats1_kernel, 64,
                    [a0, c0, W1, _vec(b1)])(x0g, qw, vf)
    m1 = s1[0] / n
    v1 = s1[1] / n - m1 * m1
    a1 = _vec(g1 / jnp.sqrt(v1 + _EPS))
    c1 = _vec(be1 - m1 * (g1 / jnp.sqrt(v1 + _EPS)))

    s2 = _edge_pass(_stats2_kernel, 128,
                    [a0, c0, W1, _vec(b1), a1, c1, W2, _vec(b2)])(x0g, qw, vf)
    m2 = s2[0] / n
    v2 = s2[1] / n - m2 * m2
    a2 = _vec(g2 / jnp.sqrt(v2 + _EPS))
    c2 = _vec(be2 - m2 * (g2 / jnp.sqrt(v2 + _EPS)))

    out = _edge_pass(_final_kernel, 128,
                     [a0, c0, W1, _vec(b1), a1, c1, W2, _vec(b2), a2, c2],
                     out_rows=_B * _M)(x0g, qw, vf)

    pos_new = pos[idx_g]
    batch_new = batch[idx_g]
    return out, pos_new, batch_new
